# baseline (speedup 1.0000x reference)
import functools

import jax
import jax.numpy as jnp
from jax import lax
from jax.experimental import pallas as pl
from jax.experimental.pallas import tpu as pltpu
from jax.experimental.pallas import tpu_sc as plsc

NUM_CORES = 2
NUM_SUBCORES = 16
LANES = 16
NW = NUM_CORES * NUM_SUBCORES


@functools.lru_cache(maxsize=None)
def _build_lookup(n_total: int, num_heads: int, d: int):
    assert n_total % NW == 0
    per_w = n_total // NW
    chunk = 1024 if per_w % 1024 == 0 else per_w
    n_chunks = per_w // chunk
    assert per_w % LANES == 0

    mesh = plsc.VectorSubcoreMesh(core_axis_name="c", subcore_axis_name="s")

    @functools.partial(
        pl.kernel,
        out_type=jax.ShapeDtypeStruct((n_total, d), jnp.float32),
        mesh=mesh,
        compiler_params=pltpu.CompilerParams(use_tc_tiling_on_sc=False),
        scratch_types=[
            pltpu.VMEM((per_w,), jnp.int32),
            pltpu.VMEM((LANES,), jnp.int32),
            pltpu.VMEM((2, chunk, d), jnp.float32),
            pltpu.SemaphoreType.DMA,
            pltpu.SemaphoreType.DMA,
            pltpu.SemaphoreType.DMA,
            pltpu.SemaphoreType.DMA,
        ],
    )
    def lookup(ids_hbm, offs_hbm, table_hbm, out_hbm,
               idx_v, offs_v, rows_v, g0, g1, s0, s1):
        wid = lax.axis_index("s") * NUM_CORES + lax.axis_index("c")
        base = wid * per_w

        pltpu.sync_copy(ids_hbm.at[pl.ds(base, per_w)], idx_v)
        for rep in range(LANES // num_heads):
            pltpu.sync_copy(offs_hbm, offs_v.at[pl.ds(rep * num_heads, num_heads)])
        offs16 = offs_v[...]

        def add_body(i, carry):
            sl = pl.ds(i * LANES, LANES)
            idx_v[sl] = idx_v[sl] + offs16
            return carry

        lax.fori_loop(0, per_w // LANES, add_body, 0)

        gsems = (g0, g1)
        ssems = (s0, s1)

        def gather(c):
            return pltpu.make_async_copy(
                table_hbm.at[idx_v.at[pl.ds(c * chunk, chunk)]],
                rows_v.at[c % 2], gsems[c % 2])

        def store(c):
            return pltpu.make_async_copy(
                rows_v.at[c % 2],
                out_hbm.at[pl.ds(base + c * chunk, chunk)], ssems[c % 2])

        gather(0).start()
        for c in range(n_chunks):
            gather(c).wait()
            if c + 1 < n_chunks:
                if c >= 1:
                    store(c - 1).wait()
                gather(c + 1).start()
            store(c).start()
        store(n_chunks - 1).wait()
        if n_chunks >= 2:
            store(n_chunks - 2).wait()

    return lookup


def kernel(input_ids, offsets, table):
    b, h = input_ids.shape
    _, d = table.shape
    ids_flat = input_ids.reshape(b * h)
    out = _build_lookup(b * h, h, d)(ids_flat, offsets, table)
    return out.reshape(b, h, d)
